# baseline (speedup 1.0000x reference)
import functools

import jax
import jax.numpy as jnp
from jax import lax
from jax.experimental import pallas as pl
from jax.experimental.pallas import tpu as pltpu
from jax.experimental.pallas import tpu_sc as plsc

NC = 2
NS = 16
NW = NC * NS
C = 64
DEGW = 128


def _mesh():
    return plsc.VectorSubcoreMesh(
        core_axis_name="c", subcore_axis_name="s", num_cores=NC, num_subcores=NS
    )


def _make_deg_kernel(npad, calloc):
    rows_per_tile = npad // NS
    assert calloc % 6 == 0 and rows_per_tile % 128 == 0
    groups = calloc // 6

    @functools.partial(
        pl.kernel,
        out_type=jax.ShapeDtypeStruct((NC, npad, DEGW), jnp.float32),
        mesh=_mesh(),
        scratch_types=[
            pltpu.VMEM((calloc, 2 * C), jnp.int32),
            pltpu.VMEM((C, DEGW), jnp.float32),
            pltpu.VMEM((C, DEGW), jnp.float32),
            pltpu.VMEM_SHARED((npad, DEGW), jnp.float32),
            pltpu.SemaphoreType.DMA,
            pltpu.SemaphoreType.DMA,
            pltpu.SemaphoreType.DMA,
            pltpu.SemaphoreType.DMA,
            pltpu.SemaphoreType.DMA,
            pltpu.SemaphoreType.DMA,
        ],
    )
    def deg_kernel(dst_hbm, out_hbm, idx_v, ones_v, zeros_v, acc_sh,
                   s0, s1, s2, s3, s4, s5):
        sems = (s0, s1, s2, s3, s4, s5)
        c = lax.axis_index("c")
        s = lax.axis_index("s")
        w = c * NS + s

        def fill(i, carry):
            for kk in range(DEGW // 16):
                ones_v[i, pl.ds(kk * 16, 16)] = jnp.full((16,), 1.0, jnp.float32)
                zeros_v[i, pl.ds(kk * 16, 16)] = jnp.zeros((16,), jnp.float32)
            return carry

        lax.fori_loop(0, C, fill, 0)

        r0 = s * rows_per_tile

        def zblk(i, carry):
            pltpu.sync_copy(zeros_v, acc_sh.at[pl.ds(r0 + i * C, C)])
            return carry

        lax.fori_loop(0, rows_per_tile // C, zblk, 0)
        plsc.subcore_barrier()

        pltpu.async_copy(dst_hbm.at[w], idx_v, s0).wait()

        def body(g, carry):
            j0 = g * 6
            for b in range(6):
                pltpu.async_copy(
                    ones_v, acc_sh.at[idx_v.at[j0 + b, pl.ds(0, C)]], sems[b],
                    add=True,
                )
            for b in range(6):
                pltpu.make_async_copy(
                    ones_v, acc_sh.at[idx_v.at[j0 + b, pl.ds(0, C)]], sems[b]
                ).wait()
            return carry

        lax.fori_loop(0, groups, body, 0)
        plsc.subcore_barrier()

        def oblk(i, carry):
            pltpu.sync_copy(
                acc_sh.at[pl.ds(r0 + i * 128, 128)],
                out_hbm.at[c, pl.ds(r0 + i * 128, 128)],
            )
            return carry

        lax.fori_loop(0, rows_per_tile // 128, oblk, 0)

    return deg_kernel


def _make_scatter_kernel(npad, chunks, calloc, d):
    rows_per_tile = npad // NS
    assert chunks % 3 == 0 and calloc >= chunks + 2
    ngroups = chunks // 3

    @functools.partial(
        pl.kernel,
        out_type=jax.ShapeDtypeStruct((NC, npad, d), jnp.float32),
        mesh=_mesh(),
        scratch_types=[
            pltpu.VMEM((calloc, 2 * C), jnp.int32),
            pltpu.VMEM((C, d), jnp.float32),
            pltpu.VMEM((C, d), jnp.float32),
            pltpu.VMEM((C, d), jnp.float32),
            pltpu.VMEM_SHARED((npad, d), jnp.float32),
            pltpu.SemaphoreType.DMA,
            pltpu.SemaphoreType.DMA,
            pltpu.SemaphoreType.DMA,
            pltpu.SemaphoreType.DMA,
            pltpu.SemaphoreType.DMA,
            pltpu.SemaphoreType.DMA,
        ],
    )
    def scatter_kernel(table_hbm, sd_hbm, out_hbm, sd_v,
                       b0, b1, b2, acc_sh, gs0, gs1, gs2, ss0, ss1, ss2):
        bufs = (b0, b1, b2)
        gsems = (gs0, gs1, gs2)
        ssems = (ss0, ss1, ss2)
        c = lax.axis_index("c")
        s = lax.axis_index("s")
        w = c * NS + s

        def zrow(i, carry):
            for kk in range(d // 16):
                b0[i, pl.ds(kk * 16, 16)] = jnp.zeros((16,), jnp.float32)
            return carry

        lax.fori_loop(0, C, zrow, 0)

        r0 = s * rows_per_tile

        def zblk(i, carry):
            pltpu.sync_copy(b0, acc_sh.at[pl.ds(r0 + i * C, C)])
            return carry

        lax.fori_loop(0, rows_per_tile // C, zblk, 0)
        plsc.subcore_barrier()

        pltpu.async_copy(sd_hbm.at[w], sd_v, gs0).wait()

        def gather_start(j, b):
            pltpu.async_copy(
                table_hbm.at[sd_v.at[j, pl.ds(C, C)]], bufs[b], gsems[b]
            )

        def gather_wait(j, b):
            pltpu.make_async_copy(
                table_hbm.at[sd_v.at[j, pl.ds(C, C)]], bufs[b], gsems[b]
            ).wait()

        def scat_start(j, b):
            pltpu.async_copy(
                bufs[b], acc_sh.at[sd_v.at[j, pl.ds(0, C)]], ssems[b], add=True
            )

        def scat_wait(j, b):
            pltpu.make_async_copy(
                bufs[b], acc_sh.at[sd_v.at[j, pl.ds(0, C)]], ssems[b]
            ).wait()

        gather_start(0, 0)
        gather_start(1, 1)
        gather_wait(0, 0)
        scat_start(0, 0)
        gather_start(2, 2)
        gather_wait(1, 1)
        scat_start(1, 1)
        scat_wait(0, 0)
        gather_start(3, 0)
        gather_wait(2, 2)
        scat_start(2, 2)
        scat_wait(1, 1)
        gather_start(4, 1)

        def body(g, carry):
            j0 = g * 3
            for i in range(3):
                gather_wait(j0 + i, i)
                scat_start(j0 + i, i)
                scat_wait(j0 + i - 1, (i + 2) % 3)
                gather_start(j0 + i + 2, (i + 2) % 3)
            return carry

        lax.fori_loop(1, ngroups, body, 0)

        scat_wait(chunks - 1, 2)
        gather_wait(chunks, 0)
        gather_wait(chunks + 1, 1)
        plsc.subcore_barrier()

        def oblk(i, carry):
            pltpu.sync_copy(
                acc_sh.at[pl.ds(r0 + i * 128, 128)],
                out_hbm.at[c, pl.ds(r0 + i * 128, 128)],
            )
            return carry

        lax.fori_loop(0, rows_per_tile // 128, oblk, 0)

    return scatter_kernel


def _elu(x):
    return jnp.where(x > 0, x, jnp.exp(x) - 1.0)


def _prep_body(x_ref, w1_ref, degp_ref, dinv_ref, hw1s_ref):
    xb = x_ref[...]
    deg = 1.0 + degp_ref[0, :, 0:1] + degp_ref[1, :, 0:1]
    dinv = lax.rsqrt(deg)
    hw = lax.dot_general(
        xb, w1_ref[...], (((0,), (0,)), ((), ())),
        preferred_element_type=jnp.float32,
    )
    hw1s_ref[...] = hw * dinv
    dinv_ref[...] = jnp.broadcast_to(dinv, dinv_ref.shape)


def _mid_body(p_ref, hw1s_ref, dinv_ref, b1_ref, w2_ref, hw2s_ref):
    acc = p_ref[0] + p_ref[1] + hw1s_ref[...]
    dinv = dinv_ref[:, 0:1]
    o = acc * dinv + b1_ref[...]
    h2 = _elu(o)
    hw2 = jnp.dot(h2, w2_ref[...], preferred_element_type=jnp.float32)
    hw2s_ref[...] = hw2 * dinv


def _fin_body(q_ref, hw2s_ref, dinv_ref, b2_ref, wfc_ref, bfc_ref, y_ref):
    acc = q_ref[0] + q_ref[1] + hw2s_ref[...]
    dinv = dinv_ref[:, 0:1]
    o = acc * dinv + b2_ref[...]
    h2 = _elu(o)
    y = jnp.dot(h2, wfc_ref[...], preferred_element_type=jnp.float32) + bfc_ref[...]
    y_ref[...] = y


def kernel(x, edge_index, W1, b1, W2, b2, Wfc, bfc):
    _, d, n = x.shape
    e = edge_index.shape[1]
    npad = ((n + NS * 128 - 1) // (NS * 128)) * (NS * 128)
    bn = 1024
    assert npad % bn == 0 and d % 16 == 0

    chunks = (((e + NW * C - 1) // (NW * C)) + 2) // 3 * 3
    calloc = (chunks + 2 + 5) // 6 * 6
    etot = chunks * C * NW
    pad_idx = jnp.full((etot - e,), n, jnp.int32)
    over = jnp.full((NW, calloc - chunks, C), n, jnp.int32)
    src_p = jnp.concatenate(
        [jnp.concatenate([edge_index[0], pad_idx]).reshape(NW, chunks, C), over], axis=1
    )
    dst_p = jnp.concatenate(
        [jnp.concatenate([edge_index[1], pad_idx]).reshape(NW, chunks, C), over], axis=1
    )
    sd_p = jnp.concatenate([dst_p, src_p], axis=2)
    x_pad = jnp.pad(x[0], ((0, 0), (0, npad - n)))

    degp = _make_deg_kernel(npad, calloc)(sd_p)

    grid = (npad // bn,)
    dinv, hw1s = pl.pallas_call(
        _prep_body,
        grid=grid,
        in_specs=[
            pl.BlockSpec((d, bn), lambda i: (0, i)),
            pl.BlockSpec((d, d), lambda i: (0, 0)),
            pl.BlockSpec((2, bn, DEGW), lambda i: (0, i, 0)),
        ],
        out_specs=[
            pl.BlockSpec((bn, 8), lambda i: (i, 0)),
            pl.BlockSpec((bn, d), lambda i: (i, 0)),
        ],
        out_shape=[
            jax.ShapeDtypeStruct((npad, 8), jnp.float32),
            jax.ShapeDtypeStruct((npad, d), jnp.float32),
        ],
    )(x_pad, W1, degp)

    p1 = _make_scatter_kernel(npad, chunks, calloc, d)(hw1s, sd_p)

    hw2s = pl.pallas_call(
        _mid_body,
        grid=grid,
        in_specs=[
            pl.BlockSpec((2, bn, d), lambda i: (0, i, 0)),
            pl.BlockSpec((bn, d), lambda i: (i, 0)),
            pl.BlockSpec((bn, 8), lambda i: (i, 0)),
            pl.BlockSpec((1, d), lambda i: (0, 0)),
            pl.BlockSpec((d, d), lambda i: (0, 0)),
        ],
        out_specs=pl.BlockSpec((bn, d), lambda i: (i, 0)),
        out_shape=jax.ShapeDtypeStruct((npad, d), jnp.float32),
    )(p1, hw1s, dinv, b1.reshape(1, d), W2)

    p2 = _make_scatter_kernel(npad, chunks, calloc, d)(hw2s, sd_p)

    y = pl.pallas_call(
        _fin_body,
        grid=grid,
        in_specs=[
            pl.BlockSpec((2, bn, d), lambda i: (0, i, 0)),
            pl.BlockSpec((bn, d), lambda i: (i, 0)),
            pl.BlockSpec((bn, 8), lambda i: (i, 0)),
            pl.BlockSpec((1, d), lambda i: (0, 0)),
            pl.BlockSpec((d, 1), lambda i: (0, 0)),
            pl.BlockSpec((1, 1), lambda i: (0, 0)),
        ],
        out_specs=pl.BlockSpec((bn, 1), lambda i: (i, 0)),
        out_shape=jax.ShapeDtypeStruct((npad, 1), jnp.float32),
    )(p2, hw2s, dinv, b2.reshape(1, d), Wfc, bfc.reshape(1, 1))

    return y[:n, 0].reshape(1, 1, 1, n)


# speedup vs baseline: 1.2611x; 1.2611x over previous
import functools

import jax
import jax.numpy as jnp
from jax import lax
from jax.experimental import pallas as pl
from jax.experimental.pallas import tpu as pltpu
from jax.experimental.pallas import tpu_sc as plsc

NC = 2
NS = 16
NW = NC * NS
C = 64
DEGW = 128


def _mesh():
    return plsc.VectorSubcoreMesh(
        core_axis_name="c", subcore_axis_name="s", num_cores=NC, num_subcores=NS
    )


def _make_deg_kernel(npad, calloc):
    rows_per_tile = npad // NS
    assert calloc % 4 == 0 and rows_per_tile % 128 == 0
    groups = calloc // 4

    @functools.partial(
        pl.kernel,
        out_type=jax.ShapeDtypeStruct((NC, npad, DEGW), jnp.float32),
        mesh=_mesh(),
        scratch_types=[
            pltpu.VMEM((calloc, 2 * C), jnp.int32),
            pltpu.VMEM((C, DEGW), jnp.float32),
            pltpu.VMEM_SHARED((npad, DEGW), jnp.float32),
            pltpu.SemaphoreType.DMA,
            pltpu.SemaphoreType.DMA,
            pltpu.SemaphoreType.DMA,
            pltpu.SemaphoreType.DMA,
        ],
    )
    def deg_kernel(dst_hbm, out_hbm, idx_v, ones_v, acc_sh, s0, s1, s2, s3):
        sems = (s0, s1, s2, s3)
        c = lax.axis_index("c")
        s = lax.axis_index("s")
        w = c * NS + s

        def fillz(i, carry):
            for kk in range(DEGW // 16):
                ones_v[i, pl.ds(kk * 16, 16)] = jnp.zeros((16,), jnp.float32)
            return carry

        lax.fori_loop(0, C, fillz, 0)

        r0 = s * rows_per_tile

        def zblk(i, carry):
            pltpu.sync_copy(ones_v.at[pl.ds(0, 64)],
                            acc_sh.at[pl.ds(r0 + i * 64, 64)])
            return carry

        lax.fori_loop(0, rows_per_tile // 64, zblk, 0)
        plsc.subcore_barrier()

        def fill1(i, carry):
            for kk in range(DEGW // 16):
                ones_v[i, pl.ds(kk * 16, 16)] = jnp.full((16,), 1.0, jnp.float32)
            return carry

        lax.fori_loop(0, C, fill1, 0)

        pltpu.async_copy(dst_hbm.at[w], idx_v, s0).wait()

        def body(g, carry):
            j0 = g * 4
            for b in range(4):
                pltpu.async_copy(
                    ones_v, acc_sh.at[idx_v.at[j0 + b, pl.ds(0, C)]], sems[b],
                    add=True,
                )
            for b in range(4):
                pltpu.make_async_copy(
                    ones_v, acc_sh.at[idx_v.at[j0 + b, pl.ds(0, C)]], sems[b]
                ).wait()
            return carry

        lax.fori_loop(0, groups, body, 0)
        plsc.subcore_barrier()

        def oblk(i, carry):
            pltpu.sync_copy(
                acc_sh.at[pl.ds(r0 + i * 128, 128)],
                out_hbm.at[c, pl.ds(r0 + i * 128, 128)],
            )
            return carry

        lax.fori_loop(0, rows_per_tile // 128, oblk, 0)

    return deg_kernel


def _make_scatter_kernel(npad, chunks, calloc, d):
    rows_per_tile = npad // NS
    assert chunks % 2 == 0 and calloc >= chunks + 2
    ngroups = chunks // 2

    @functools.partial(
        pl.kernel,
        out_type=jax.ShapeDtypeStruct((NC, npad, d), jnp.float32),
        mesh=_mesh(),
        scratch_types=[
            pltpu.VMEM((calloc, 2 * C), jnp.int32),
            pltpu.VMEM((C, d), jnp.float32),
            pltpu.VMEM((C, d), jnp.float32),
            pltpu.VMEM_SHARED((npad, d), jnp.float32),
            pltpu.SemaphoreType.DMA,
            pltpu.SemaphoreType.DMA,
        ],
    )
    def scatter_kernel(table_hbm, sd_hbm, out_hbm, sd_v,
                       b0, b1, acc_sh, gs0, gs1):
        c = lax.axis_index("c")
        s = lax.axis_index("s")
        w = c * NS + s

        def zrow(i, carry):
            for kk in range(d // 16):
                b0[i, pl.ds(kk * 16, 16)] = jnp.zeros((16,), jnp.float32)
            return carry

        lax.fori_loop(0, C, zrow, 0)

        r0 = s * rows_per_tile

        def zblk(i, carry):
            pltpu.sync_copy(b0.at[pl.ds(0, 64)],
                            acc_sh.at[pl.ds(r0 + i * 64, 64)])
            return carry

        lax.fori_loop(0, rows_per_tile // 64, zblk, 0)
        plsc.subcore_barrier()

        pltpu.async_copy(sd_hbm.at[w], sd_v, gs0).wait()

        def gather_make(j, buf, sem):
            return pltpu.make_async_copy(
                table_hbm.at[sd_v.at[j, pl.ds(C, C)]], buf, sem
            )

        def scat_sync(j, buf):
            pltpu.sync_copy(buf, acc_sh.at[sd_v.at[j, pl.ds(0, C)]], add=True)

        gather_make(0, b0, gs0).start()

        def body(g, carry):
            j = g * 2
            gather_make(j, b0, gs0).wait()
            gather_make(j + 1, b1, gs1).start()
            scat_sync(j, b0)
            gather_make(j + 1, b1, gs1).wait()
            gather_make(j + 2, b0, gs0).start()
            scat_sync(j + 1, b1)
            return carry

        lax.fori_loop(0, ngroups, body, 0)

        gather_make(chunks, b0, gs0).wait()
        plsc.subcore_barrier()

        def oblk(i, carry):
            pltpu.sync_copy(
                acc_sh.at[pl.ds(r0 + i * 128, 128)],
                out_hbm.at[c, pl.ds(r0 + i * 128, 128)],
            )
            return carry

        lax.fori_loop(0, rows_per_tile // 128, oblk, 0)

    return scatter_kernel


def _elu(x):
    return jnp.where(x > 0, x, jnp.exp(x) - 1.0)


def _prep_body(x_ref, w1_ref, degp_ref, dinv_ref, hw1s_ref):
    xb = x_ref[...]
    deg = 1.0 + degp_ref[0, :, 0:1] + degp_ref[1, :, 0:1]
    dinv = lax.rsqrt(deg)
    hw = lax.dot_general(
        xb, w1_ref[...], (((0,), (0,)), ((), ())),
        preferred_element_type=jnp.float32,
    )
    hw1s_ref[...] = hw * dinv
    dinv_ref[...] = jnp.broadcast_to(dinv, dinv_ref.shape)


def _mid_body(p_ref, hw1s_ref, dinv_ref, b1_ref, w2_ref, hw2s_ref):
    acc = p_ref[0] + p_ref[1] + hw1s_ref[...]
    dinv = dinv_ref[:, 0:1]
    o = acc * dinv + b1_ref[...]
    h2 = _elu(o)
    hw2 = jnp.dot(h2, w2_ref[...], preferred_element_type=jnp.float32)
    hw2s_ref[...] = hw2 * dinv


def _fin_body(q_ref, hw2s_ref, dinv_ref, b2_ref, wfc_ref, bfc_ref, y_ref):
    acc = q_ref[0] + q_ref[1] + hw2s_ref[...]
    dinv = dinv_ref[:, 0:1]
    o = acc * dinv + b2_ref[...]
    h2 = _elu(o)
    y = jnp.dot(h2, wfc_ref[...], preferred_element_type=jnp.float32) + bfc_ref[...]
    y_ref[...] = y


def kernel(x, edge_index, W1, b1, W2, b2, Wfc, bfc):
    _, d, n = x.shape
    e = edge_index.shape[1]
    npad = ((n + NS * 128 - 1) // (NS * 128)) * (NS * 128)
    bn = 1024
    assert npad % bn == 0 and d % 16 == 0

    chunks = (((e + NW * C - 1) // (NW * C)) + 1) // 2 * 2
    calloc = (chunks + 2 + 3) // 4 * 4
    etot = chunks * C * NW
    pad_idx = jnp.full((etot - e,), n, jnp.int32)
    over = jnp.full((NW, calloc - chunks, C), n, jnp.int32)
    src_p = jnp.concatenate(
        [jnp.concatenate([edge_index[0], pad_idx]).reshape(NW, chunks, C), over], axis=1
    )
    dst_p = jnp.concatenate(
        [jnp.concatenate([edge_index[1], pad_idx]).reshape(NW, chunks, C), over], axis=1
    )
    sd_p = jnp.concatenate([dst_p, src_p], axis=2)
    x_pad = jnp.pad(x[0], ((0, 0), (0, npad - n)))

    degp = _make_deg_kernel(npad, calloc)(sd_p)

    grid = (npad // bn,)
    dinv, hw1s = pl.pallas_call(
        _prep_body,
        grid=grid,
        in_specs=[
            pl.BlockSpec((d, bn), lambda i: (0, i)),
            pl.BlockSpec((d, d), lambda i: (0, 0)),
            pl.BlockSpec((2, bn, DEGW), lambda i: (0, i, 0)),
        ],
        out_specs=[
            pl.BlockSpec((bn, 8), lambda i: (i, 0)),
            pl.BlockSpec((bn, d), lambda i: (i, 0)),
        ],
        out_shape=[
            jax.ShapeDtypeStruct((npad, 8), jnp.float32),
            jax.ShapeDtypeStruct((npad, d), jnp.float32),
        ],
    )(x_pad, W1, degp)

    p1 = _make_scatter_kernel(npad, chunks, calloc, d)(hw1s, sd_p)

    hw2s = pl.pallas_call(
        _mid_body,
        grid=grid,
        in_specs=[
            pl.BlockSpec((2, bn, d), lambda i: (0, i, 0)),
            pl.BlockSpec((bn, d), lambda i: (i, 0)),
            pl.BlockSpec((bn, 8), lambda i: (i, 0)),
            pl.BlockSpec((1, d), lambda i: (0, 0)),
            pl.BlockSpec((d, d), lambda i: (0, 0)),
        ],
        out_specs=pl.BlockSpec((bn, d), lambda i: (i, 0)),
        out_shape=jax.ShapeDtypeStruct((npad, d), jnp.float32),
    )(p1, hw1s, dinv, b1.reshape(1, d), W2)

    p2 = _make_scatter_kernel(npad, chunks, calloc, d)(hw2s, sd_p)

    y = pl.pallas_call(
        _fin_body,
        grid=grid,
        in_specs=[
            pl.BlockSpec((2, bn, d), lambda i: (0, i, 0)),
            pl.BlockSpec((bn, d), lambda i: (i, 0)),
            pl.BlockSpec((bn, 8), lambda i: (i, 0)),
            pl.BlockSpec((1, d), lambda i: (0, 0)),
            pl.BlockSpec((d, 1), lambda i: (0, 0)),
            pl.BlockSpec((1, 1), lambda i: (0, 0)),
        ],
        out_specs=pl.BlockSpec((bn, 1), lambda i: (i, 0)),
        out_shape=jax.ShapeDtypeStruct((npad, 1), jnp.float32),
    )(p2, hw2s, dinv, b2.reshape(1, d), Wfc, bfc.reshape(1, 1))

    return y[:n, 0].reshape(1, 1, 1, n)
